# Initial kernel scaffold; baseline (speedup 1.0000x reference)
import jax
import jax.numpy as jnp
from jax.experimental import pallas as pl


def kernel(x, emb_weight):
    raise NotImplementedError("write your pallas kernel here")


# speedup vs baseline: 1.3671x; 1.3671x over previous
import jax
import jax.numpy as jnp
from jax.experimental import pallas as pl
from jax.experimental.pallas import tpu as pltpu


def _add_body(x_ref, e_ref, o_ref):
    o_ref[...] = x_ref[...] + e_ref[...][None]


def kernel(x, emb_weight):
    B, S, D = x.shape
    SCHUNK = 512
    grid = (S // SCHUNK, B)

    out = pl.pallas_call(
        _add_body,
        grid=grid,
        in_specs=[
            pl.BlockSpec((1, SCHUNK, D), lambda s, b: (b, s, 0)),
            pl.BlockSpec((SCHUNK, D), lambda s, b: (s, 0)),
        ],
        out_specs=pl.BlockSpec((1, SCHUNK, D), lambda s, b: (b, s, 0)),
        out_shape=jax.ShapeDtypeStruct((B, S, D), x.dtype),
    )(x, emb_weight[:S])
    return out
